# baseline (speedup 1.0000x reference)
import functools

import jax
import jax.numpy as jnp
from jax.experimental import pallas as pl
from jax.experimental.pallas import tpu as pltpu

N, D, H, O, E, TOPK = 2048, 1024, 2048, 1024, 8, 2
TN = 256
NT = N // TN
EP = 128


def _gating_body(x_ref, wg_ref, bg_ref, probs_ref, selw_ref, loss_ref, acc_ref):
    i = pl.program_id(0)
    xt = x_ref[...]
    logits = jax.lax.dot_general(
        xt, wg_ref[...], (((1,), (0,)), ((), ())),
        preferred_element_type=jnp.float32) + bg_ref[...]
    col = jax.lax.broadcasted_iota(jnp.int32, (TN, EP), 1)
    valid = col < E
    neg = jnp.float32(-jnp.inf)
    logits = jnp.where(valid, logits, neg)
    m = jnp.max(logits, axis=1, keepdims=True)
    ex = jnp.exp(logits - m)
    s = jnp.sum(ex, axis=1, keepdims=True)
    probs = ex / s
    probs_ref[...] = probs

    p1 = jnp.max(probs, axis=1, keepdims=True)
    i1 = jnp.min(jnp.where((probs == p1) & valid, col, EP), axis=1, keepdims=True)
    one1 = col == i1
    probs_m = jnp.where(one1, -1.0, probs)
    p2 = jnp.max(probs_m, axis=1, keepdims=True)
    i2 = jnp.min(jnp.where((probs_m == p2) & valid, col, EP), axis=1, keepdims=True)
    one2 = col == i2
    denom = p1 + p2
    selw_ref[...] = jnp.where(one1, p1 / denom, 0.0) + jnp.where(one2, p2 / denom, 0.0)

    part = jnp.sum(probs, axis=0, keepdims=True)
    @pl.when(i == 0)
    def _():
        acc_ref[...] = part
    @pl.when(i > 0)
    def _():
        acc_ref[...] += part

    @pl.when(i == NT - 1)
    def _():
        usage = acc_ref[...] / N
        lane = jax.lax.broadcasted_iota(jnp.int32, (1, EP), 1)
        uni = jnp.float32(1.0 / E)
        term = uni * (jnp.log(uni) - jnp.log(usage + 1e-8))
        loss_ref[...] = jnp.sum(jnp.where(lane < E, term, 0.0), axis=1, keepdims=True) * 0.01


def _ffn_body(x_ref, w1_ref, b1_ref, w2_ref, b2_ref, w3_ref, b3_ref, selw_ref,
              out_ref):
    e = pl.program_id(0)
    i = pl.program_id(1)
    xt = x_ref[...].astype(jnp.bfloat16)
    h1 = jax.lax.dot_general(
        xt, w1_ref[0], (((1,), (1,)), ((), ())),
        preferred_element_type=jnp.float32) + b1_ref[0]
    h1 = jnp.maximum(h1, 0.0).astype(jnp.bfloat16)
    h2 = jax.lax.dot_general(
        h1, w2_ref[0], (((1,), (1,)), ((), ())),
        preferred_element_type=jnp.float32) + b2_ref[0]
    h2 = jnp.maximum(h2, 0.0).astype(jnp.bfloat16)
    y = jax.lax.dot_general(
        h2, w3_ref[0], (((1,), (1,)), ((), ())),
        preferred_element_type=jnp.float32) + b3_ref[0]
    y = y * selw_ref[0]
    sl = pl.ds(i * TN, TN)
    @pl.when(e == 0)
    def _():
        out_ref[sl, :] = y
    @pl.when(e > 0)
    def _():
        out_ref[sl, :] += y


def kernel(x, Wg, bg, W1, b1, W2, b2, W3, b3):
    wgp = jnp.zeros((D, EP), jnp.float32).at[:, :E].set(Wg.T)
    bgp = jnp.zeros((1, EP), jnp.float32).at[0, :E].set(bg)
    probs_p, selw_p, loss2 = pl.pallas_call(
        _gating_body,
        grid=(NT,),
        in_specs=[
            pl.BlockSpec((TN, D), lambda i: (i, 0)),
            pl.BlockSpec((D, EP), lambda i: (0, 0)),
            pl.BlockSpec((1, EP), lambda i: (0, 0)),
        ],
        out_specs=[
            pl.BlockSpec((TN, EP), lambda i: (i, 0)),
            pl.BlockSpec((TN, EP), lambda i: (i, 0)),
            pl.BlockSpec((1, 1), lambda i: (0, 0)),
        ],
        out_shape=[
            jax.ShapeDtypeStruct((N, EP), jnp.float32),
            jax.ShapeDtypeStruct((N, EP), jnp.float32),
            jax.ShapeDtypeStruct((1, 1), jnp.float32),
        ],
        scratch_shapes=[pltpu.VMEM((1, EP), jnp.float32)],
    )(x, wgp, bgp)

    gate_probs = probs_p[:, :E]
    loss = loss2.reshape(())

    selw = selw_p[:, :E].T.reshape(E * NT, TN, 1)
    w1b = W1.astype(jnp.bfloat16)
    w2b = W2.astype(jnp.bfloat16)
    w3b = W3.astype(jnp.bfloat16)
    b1r = b1.reshape(E, 1, H)
    b2r = b2.reshape(E, 1, H)
    b3r = b3.reshape(E, 1, O)
    final = pl.pallas_call(
        _ffn_body,
        grid=(E, NT),
        in_specs=[
            pl.BlockSpec((TN, D), lambda e, i: (i, 0)),
            pl.BlockSpec((1, H, D), lambda e, i: (e, 0, 0)),
            pl.BlockSpec((1, 1, H), lambda e, i: (e, 0, 0)),
            pl.BlockSpec((1, H, H), lambda e, i: (e, 0, 0)),
            pl.BlockSpec((1, 1, H), lambda e, i: (e, 0, 0)),
            pl.BlockSpec((1, O, H), lambda e, i: (e, 0, 0)),
            pl.BlockSpec((1, 1, O), lambda e, i: (e, 0, 0)),
            pl.BlockSpec((1, TN, 1), lambda e, i: (e * NT + i, 0, 0)),
        ],
        out_specs=pl.BlockSpec((N, O), lambda e, i: (0, 0)),
        out_shape=jax.ShapeDtypeStruct((N, O), jnp.float32),
    )(x, w1b, b1r, w2b, b2r, w3b, b3r, selw)

    return (final, loss, gate_probs)
